# baseline (speedup 1.0000x reference)
import functools

import jax
import jax.numpy as jnp
from jax import lax
from jax.experimental import pallas as pl
from jax.experimental.pallas import tpu as pltpu
from jax.experimental.pallas import tpu_sc as plsc


def kernel(x, table):
    seq_len = x.shape[1]
    embed_dim = table.shape[1]
    info = plsc.get_sparse_core_info()
    num_workers = info.num_cores * info.num_subcores
    assert seq_len % num_workers == 0
    rows_per_worker = seq_len // num_workers
    mesh = plsc.VectorSubcoreMesh(core_axis_name="c", subcore_axis_name="s")

    @functools.partial(
        pl.kernel,
        mesh=mesh,
        out_type=jax.ShapeDtypeStruct((seq_len, embed_dim), table.dtype),
        scratch_types=[
            pltpu.VMEM((rows_per_worker, embed_dim), table.dtype),
        ],
    )
    def _copy(table_hbm, out_hbm, vbuf):
        wid = lax.axis_index("s") * info.num_cores + lax.axis_index("c")
        base = wid * rows_per_worker
        pltpu.sync_copy(table_hbm.at[pl.ds(base, rows_per_worker)], vbuf)
        pltpu.sync_copy(vbuf, out_hbm.at[pl.ds(base, rows_per_worker)])

    return _copy(table)
